# Initial kernel scaffold; baseline (speedup 1.0000x reference)
import jax
import jax.numpy as jnp
from jax.experimental import pallas as pl


def kernel(objects, relations, senders, receivers, re_W1, re_b1, re_W2, re_b2, oe_W1, oe_b1, oe_W2, oe_b2, ea_W1, ea_b1, ea_W2, ea_b2, od_W1, od_b1, od_W2, od_b2):
    raise NotImplementedError("write your pallas kernel here")


# speedup vs baseline: 3.0392x; 3.0392x over previous
import functools

import jax
import jax.numpy as jnp
from jax import lax
from jax.experimental import pallas as pl
from jax.experimental.pallas import tpu as pltpu
from jax.experimental.pallas import tpu_sc as plsc

N = 10000
E = 160000
D = 128
H = 256

_NC = 2
_NS = 16
_NW = _NC * _NS

_CH = 200
_EPW = E // _NW
_EPT = E // _NC // _NS
_NPT = 624
_NTAIL = N - _NS * _NPT


def _sc_gather_body(table_hbm, s_hbm, r_hbm, out_hbm,
                    s_v, r_v, rows_s, rows_r, sem):
    wid = lax.axis_index("s") * _NC + lax.axis_index("c")
    base = wid * _EPW

    def body(j, _):
        off = base + j * _CH
        pltpu.sync_copy(s_hbm.at[pl.ds(off, _CH)], s_v)
        pltpu.sync_copy(r_hbm.at[pl.ds(off, _CH)], r_v)
        cs = pltpu.async_copy(table_hbm.at[s_v], rows_s, sem)
        cr = pltpu.async_copy(table_hbm.at[r_v], rows_r, sem)
        cs.wait()
        cr.wait()
        pltpu.sync_copy(rows_s, out_hbm.at[0, pl.ds(off, _CH)])
        pltpu.sync_copy(rows_r, out_hbm.at[1, pl.ds(off, _CH)])
        return 0

    lax.fori_loop(0, _EPW // _CH, body, 0)


def _sc_gather(table, senders, receivers):
    kfn = functools.partial(
        pl.kernel,
        out_type=jax.ShapeDtypeStruct((2, E, D), jnp.float32),
        mesh=plsc.VectorSubcoreMesh(core_axis_name="c", subcore_axis_name="s"),
        scratch_types=[
            pltpu.VMEM((_CH,), jnp.int32),
            pltpu.VMEM((_CH,), jnp.int32),
            pltpu.VMEM((_CH, D), jnp.float32),
            pltpu.VMEM((_CH, D), jnp.float32),
            pltpu.SemaphoreType.DMA,
        ],
    )(_sc_gather_body)
    return kfn(table, senders, receivers)


def _sc_scatter_body(e_hbm, r_hbm, z_hbm, out_hbm, r_v, rows_v, acc_sh, sem):
    cid = lax.axis_index("c")
    sid = lax.axis_index("s")
    nbase = sid * _NPT
    pltpu.sync_copy(z_hbm.at[pl.ds(nbase, _NPT)], acc_sh.at[pl.ds(nbase, _NPT)])

    @pl.when(sid == _NS - 1)
    def _():
        pltpu.sync_copy(z_hbm.at[pl.ds(_NS * _NPT, _NTAIL)],
                        acc_sh.at[pl.ds(_NS * _NPT, _NTAIL)])

    plsc.subcore_barrier()

    base = cid * (E // _NC) + sid * _EPT

    def body(j, _):
        off = base + j * _CH
        pltpu.sync_copy(r_hbm.at[pl.ds(off, _CH)], r_v)
        pltpu.sync_copy(e_hbm.at[pl.ds(off, _CH)], rows_v)
        pltpu.sync_copy(rows_v, acc_sh.at[r_v], add=True)
        return 0

    lax.fori_loop(0, _EPT // _CH, body, 0)
    plsc.subcore_barrier()
    pltpu.sync_copy(acc_sh.at[pl.ds(nbase, _NPT)],
                    out_hbm.at[cid, pl.ds(nbase, _NPT)])

    @pl.when(sid == _NS - 1)
    def _():
        pltpu.sync_copy(acc_sh.at[pl.ds(_NS * _NPT, _NTAIL)],
                        out_hbm.at[cid, pl.ds(_NS * _NPT, _NTAIL)])


def _sc_scatter(e_hat, receivers, zeros_nd):
    kfn = functools.partial(
        pl.kernel,
        out_type=jax.ShapeDtypeStruct((2, N, D), jnp.float32),
        mesh=plsc.VectorSubcoreMesh(core_axis_name="c", subcore_axis_name="s"),
        scratch_types=[
            pltpu.VMEM((_CH,), jnp.int32),
            pltpu.VMEM((_CH, D), jnp.float32),
            pltpu.VMEM_SHARED((N, D), jnp.float32),
            pltpu.SemaphoreType.DMA,
        ],
    )(_sc_scatter_body)
    return kfn(e_hat, receivers, zeros_nd)


_BE = 2000
_BN = 2000


def _dot(a, b):
    return jax.lax.dot_general(a, b, (((1,), (0,)), ((), ())),
                               preferred_element_type=jnp.float32)


def _full(shape):
    return pl.BlockSpec(shape, lambda i: (0,) * len(shape))


def _pass0_body(g_ref, rel_ref, w1_ref, b1_ref, w2_ref, b2_ref, out_ref):
    vs = g_ref[0]
    vr = g_ref[1]
    h = _dot(vs, w1_ref[0:D]) + _dot(vr, w1_ref[D:2 * D])
    h = h + _dot(rel_ref[...], w1_ref[2 * D:3 * D]) + b1_ref[...]
    h = jnp.maximum(h, 0.0)
    renc = _dot(h, w2_ref[...]) + b2_ref[...]
    out_ref[...] = _dot(renc, w1_ref[0:D]) + b1_ref[...]


def _pass0(g0, rel, re_W1, re_b1, re_W2, re_b2):
    return pl.pallas_call(
        _pass0_body,
        grid=(E // _BE,),
        in_specs=[
            pl.BlockSpec((2, _BE, D), lambda i: (0, i, 0)),
            pl.BlockSpec((_BE, D), lambda i: (i, 0)),
            _full((3 * D, H)),
            _full((1, H)),
            _full((H, D)),
            _full((1, D)),
        ],
        out_specs=pl.BlockSpec((_BE, H), lambda i: (i, 0)),
        out_shape=jax.ShapeDtypeStruct((E, H), jnp.float32),
    )(g0, rel, re_W1, re_b1.reshape(1, H), re_W2, re_b2.reshape(1, D))


def _edge_first_body(rp_ref, w2_ref, b2_ref, out_ref):
    h = jnp.maximum(rp_ref[...], 0.0)
    out_ref[...] = _dot(h, w2_ref[...]) + b2_ref[...]


def _edge_first(rel_part, re_W2, re_b2):
    return pl.pallas_call(
        _edge_first_body,
        grid=(E // _BE,),
        in_specs=[
            pl.BlockSpec((_BE, H), lambda i: (i, 0)),
            _full((H, D)),
            _full((1, D)),
        ],
        out_specs=pl.BlockSpec((_BE, D), lambda i: (i, 0)),
        out_shape=jax.ShapeDtypeStruct((E, D), jnp.float32),
    )(rel_part, re_W2, re_b2.reshape(1, D))


def _edge_body(rp_ref, g_ref, w1_ref, w2_ref, b2_ref, out_ref):
    h = rp_ref[...] + _dot(g_ref[0], w1_ref[0:D]) + _dot(g_ref[1], w1_ref[D:2 * D])
    h = jnp.maximum(h, 0.0)
    out_ref[...] = _dot(h, w2_ref[...]) + b2_ref[...]


def _edge(rel_part, g, re_W1, re_W2, re_b2):
    return pl.pallas_call(
        _edge_body,
        grid=(E // _BE,),
        in_specs=[
            pl.BlockSpec((_BE, H), lambda i: (i, 0)),
            pl.BlockSpec((2, _BE, D), lambda i: (0, i, 0)),
            _full((2 * D, H)),
            _full((H, D)),
            _full((1, D)),
        ],
        out_specs=pl.BlockSpec((_BE, D), lambda i: (i, 0)),
        out_shape=jax.ShapeDtypeStruct((E, D), jnp.float32),
    )(rel_part, g, re_W1[D:3 * D], re_W2, re_b2.reshape(1, D))


def _prologue_body(obj_ref, ow1_ref, ob1_ref, ow2_ref, ob2_ref,
                   ea_w1a_ref, ea_b1_ref, out_ref):
    h = jnp.maximum(_dot(obj_ref[...], ow1_ref[...]) + ob1_ref[...], 0.0)
    enc = _dot(h, ow2_ref[...]) + ob2_ref[...]
    out_ref[...] = _dot(enc, ea_w1a_ref[...]) + ea_b1_ref[...]


def _prologue(obj, oe_W1, oe_b1, oe_W2, oe_b2, ea_W1, ea_b1):
    return pl.pallas_call(
        _prologue_body,
        grid=(N // _BN,),
        in_specs=[
            pl.BlockSpec((_BN, D), lambda i: (i, 0)),
            _full((D, H)),
            _full((1, H)),
            _full((H, D)),
            _full((1, D)),
            _full((D, H)),
            _full((1, H)),
        ],
        out_specs=pl.BlockSpec((_BN, H), lambda i: (i, 0)),
        out_shape=jax.ShapeDtypeStruct((N, H), jnp.float32),
    )(obj, oe_W1, oe_b1.reshape(1, H), oe_W2, oe_b2.reshape(1, D),
      ea_W1[0:D], ea_b1.reshape(1, H))


def _node_body(op_ref, agg_ref, v_ref, w1_ref, w2_ref, b2_ref, out_ref,
               *, with_v):
    agg = agg_ref[0] + agg_ref[1]
    h = op_ref[...] + _dot(agg, w1_ref[0:D])
    if with_v:
        h = h + _dot(v_ref[...], w1_ref[D:2 * D])
    h = jnp.maximum(h, 0.0)
    out_ref[...] = _dot(h, w2_ref[...]) + b2_ref[...]


def _node(obj_part, agg, v, ea_W1, ea_W2, ea_b2, with_v):
    return pl.pallas_call(
        functools.partial(_node_body, with_v=with_v),
        grid=(N // _BN,),
        in_specs=[
            pl.BlockSpec((_BN, H), lambda i: (i, 0)),
            pl.BlockSpec((2, _BN, D), lambda i: (0, i, 0)),
            pl.BlockSpec((_BN, D), lambda i: (i, 0)),
            _full((2 * D, H)),
            _full((H, D)),
            _full((1, D)),
        ],
        out_specs=pl.BlockSpec((_BN, D), lambda i: (i, 0)),
        out_shape=jax.ShapeDtypeStruct((N, D), jnp.float32),
    )(obj_part, agg, v, ea_W1[D:3 * D], ea_W2, ea_b2.reshape(1, D))


def _decode_body(v_ref, w1_ref, b1_ref, w2_ref, b2_ref, out_ref):
    h = jnp.maximum(_dot(v_ref[...], w1_ref[...]) + b1_ref[...], 0.0)
    out_ref[...] = _dot(h, w2_ref[...]) + b2_ref[...]


def _decode(v, od_W1, od_b1, od_W2, od_b2):
    return pl.pallas_call(
        _decode_body,
        grid=(N // _BN,),
        in_specs=[
            pl.BlockSpec((_BN, D), lambda i: (i, 0)),
            _full((D, H)),
            _full((1, H)),
            _full((H, D)),
            _full((1, D)),
        ],
        out_specs=pl.BlockSpec((_BN, D), lambda i: (i, 0)),
        out_shape=jax.ShapeDtypeStruct((N, D), jnp.float32),
    )(v, od_W1, od_b1.reshape(1, H), od_W2, od_b2.reshape(1, D))


def kernel(objects, relations, senders, receivers,
           re_W1, re_b1, re_W2, re_b2,
           oe_W1, oe_b1, oe_W2, oe_b2,
           ea_W1, ea_b1, ea_W2, ea_b2,
           od_W1, od_b1, od_W2, od_b2):
    assert objects.shape == (1, N, D) and relations.shape == (1, E, D)
    obj = objects[0]
    rel = relations[0]

    obj_part = _prologue(obj, oe_W1, oe_b1, oe_W2, oe_b2, ea_W1, ea_b1)
    g0 = _sc_gather(obj, senders, receivers)
    rel_part = _pass0(g0, rel, re_W1, re_b1, re_W2, re_b2)

    zeros_nd = jnp.zeros((N, D), jnp.float32)

    e_hat = _edge_first(rel_part, re_W2, re_b2)
    agg = _sc_scatter(e_hat, receivers, zeros_nd)
    v = _node(obj_part, agg, zeros_nd, ea_W1, ea_W2, ea_b2, with_v=False)

    for _ in range(2):
        g = _sc_gather(v, senders, receivers)
        e_hat = _edge(rel_part, g, re_W1, re_W2, re_b2)
        agg = _sc_scatter(e_hat, receivers, zeros_nd)
        v = _node(obj_part, agg, v, ea_W1, ea_W2, ea_b2, with_v=True)

    out = _decode(v, od_W1, od_b1, od_W2, od_b2)
    return out[None]


# speedup vs baseline: 3.3414x; 1.0994x over previous
import functools

import jax
import jax.numpy as jnp
from jax import lax
from jax.experimental import pallas as pl
from jax.experimental.pallas import tpu as pltpu
from jax.experimental.pallas import tpu_sc as plsc

N = 10000
E = 160000
D = 128
H = 256

_NC = 2
_NS = 16
_NW = _NC * _NS

_CH = 200
_CHS = 40
_EPW = E // _NW
_EPT = E // _NC // _NS
_NPT = 624
_NTAIL = N - _NS * _NPT


def _sc_gather_body(table_hbm, s_hbm, r_hbm, out_hbm,
                    s_all, r_all, rs0, rr0, rs1, rr1, sg0, sg1, sw0, sw1):
    wid = lax.axis_index("s") * _NC + lax.axis_index("c")
    base = wid * _EPW
    n_ch = _EPW // _CH

    pltpu.sync_copy(s_hbm.at[pl.ds(base, _EPW)], s_all)
    pltpu.sync_copy(r_hbm.at[pl.ds(base, _EPW)], r_all)

    def start_gather(j, rs, rr, sg):
        off = j * _CH
        pltpu.async_copy(table_hbm.at[s_all.at[pl.ds(off, _CH)]], rs, sg)
        pltpu.async_copy(table_hbm.at[r_all.at[pl.ds(off, _CH)]], rr, sg)

    def wait_gather(rs, rr, sg):
        pltpu.make_async_copy(table_hbm.at[pl.ds(0, _CH)], rs, sg).wait()
        pltpu.make_async_copy(table_hbm.at[pl.ds(0, _CH)], rr, sg).wait()

    def start_wb(j, rs, rr, sw):
        off = base + j * _CH
        pltpu.async_copy(rs, out_hbm.at[0, pl.ds(off, _CH)], sw)
        pltpu.async_copy(rr, out_hbm.at[1, pl.ds(off, _CH)], sw)

    def wait_wb(rs, rr, sw):
        pltpu.make_async_copy(rs, out_hbm.at[0, pl.ds(base, _CH)], sw).wait()
        pltpu.make_async_copy(rr, out_hbm.at[1, pl.ds(base, _CH)], sw).wait()

    def _iter(j, rs, rr, sg, sw, ors, orr, osg, osw):
        @pl.when(j + 1 < n_ch)
        def _():
            @pl.when(j >= 1)
            def _():
                wait_wb(ors, orr, osw)
            start_gather(j + 1, ors, orr, osg)
        wait_gather(rs, rr, sg)
        start_wb(j, rs, rr, sw)

    start_gather(0, rs0, rr0, sg0)

    def body(j, _):
        @pl.when(j % 2 == 0)
        def _():
            _iter(j, rs0, rr0, sg0, sw0, rs1, rr1, sg1, sw1)

        @pl.when(j % 2 == 1)
        def _():
            _iter(j, rs1, rr1, sg1, sw1, rs0, rr0, sg0, sw0)

        return 0

    lax.fori_loop(0, n_ch, body, 0)
    wait_wb(rs0, rr0, sw0)
    wait_wb(rs1, rr1, sw1)


def _sc_gather(table, senders, receivers):
    kfn = functools.partial(
        pl.kernel,
        out_type=jax.ShapeDtypeStruct((2, E, D), jnp.float32),
        mesh=plsc.VectorSubcoreMesh(core_axis_name="c", subcore_axis_name="s"),
        scratch_types=[
            pltpu.VMEM((_EPW,), jnp.int32),
            pltpu.VMEM((_EPW,), jnp.int32),
            pltpu.VMEM((_CH, D), jnp.float32),
            pltpu.VMEM((_CH, D), jnp.float32),
            pltpu.VMEM((_CH, D), jnp.float32),
            pltpu.VMEM((_CH, D), jnp.float32),
            pltpu.SemaphoreType.DMA,
            pltpu.SemaphoreType.DMA,
            pltpu.SemaphoreType.DMA,
            pltpu.SemaphoreType.DMA,
        ],
    )(_sc_gather_body)
    return kfn(table, senders, receivers)


def _sc_scatter_body(e_hbm, r_hbm, z_hbm, out_hbm,
                     rv0, ev0, rv1, ev1, acc_sh, sl0, sl1, ss0, ss1):
    cid = lax.axis_index("c")
    sid = lax.axis_index("s")
    nbase = sid * _NPT
    base = cid * (E // _NC) + sid * _EPT
    n_ch = _EPT // _CHS

    def start_load(j, rv, ev, sl):
        off = base + j * _CHS
        pltpu.async_copy(r_hbm.at[pl.ds(off, _CHS)], rv, sl)
        pltpu.async_copy(e_hbm.at[pl.ds(off, _CHS)], ev, sl)

    def wait_load(rv, ev, sl):
        pltpu.make_async_copy(r_hbm.at[pl.ds(0, _CHS)], rv, sl).wait()
        pltpu.make_async_copy(e_hbm.at[pl.ds(0, _CHS)], ev, sl).wait()

    def start_scat(rv, ev, ss):
        pltpu.async_copy(ev, acc_sh.at[rv], ss, add=True)

    def wait_scat(rv, ev, ss):
        pltpu.make_async_copy(ev, acc_sh.at[rv], ss).wait()

    start_load(0, rv0, ev0, sl0)

    pltpu.sync_copy(z_hbm.at[pl.ds(nbase, _NPT)], acc_sh.at[pl.ds(nbase, _NPT)])

    @pl.when(sid == _NS - 1)
    def _():
        pltpu.sync_copy(z_hbm.at[pl.ds(_NS * _NPT, _NTAIL)],
                        acc_sh.at[pl.ds(_NS * _NPT, _NTAIL)])

    plsc.subcore_barrier()

    def _iter(j, rv, ev, sl, ss, orv, oev, osl, oss):
        @pl.when(j + 1 < n_ch)
        def _():
            @pl.when(j >= 1)
            def _():
                wait_scat(orv, oev, oss)
            start_load(j + 1, orv, oev, osl)
        wait_load(rv, ev, sl)
        start_scat(rv, ev, ss)

    def body(j, _):
        @pl.when(j % 2 == 0)
        def _():
            _iter(j, rv0, ev0, sl0, ss0, rv1, ev1, sl1, ss1)

        @pl.when(j % 2 == 1)
        def _():
            _iter(j, rv1, ev1, sl1, ss1, rv0, ev0, sl0, ss0)

        return 0

    lax.fori_loop(0, n_ch, body, 0)
    wait_scat(rv0, ev0, ss0)
    wait_scat(rv1, ev1, ss1)
    plsc.subcore_barrier()
    pltpu.sync_copy(acc_sh.at[pl.ds(nbase, _NPT)],
                    out_hbm.at[cid, pl.ds(nbase, _NPT)])

    @pl.when(sid == _NS - 1)
    def _():
        pltpu.sync_copy(acc_sh.at[pl.ds(_NS * _NPT, _NTAIL)],
                        out_hbm.at[cid, pl.ds(_NS * _NPT, _NTAIL)])


def _sc_scatter(e_hat, receivers, zeros_nd):
    kfn = functools.partial(
        pl.kernel,
        out_type=jax.ShapeDtypeStruct((2, N, D), jnp.float32),
        mesh=plsc.VectorSubcoreMesh(core_axis_name="c", subcore_axis_name="s"),
        scratch_types=[
            pltpu.VMEM((_CHS,), jnp.int32),
            pltpu.VMEM((_CHS, D), jnp.float32),
            pltpu.VMEM((_CHS,), jnp.int32),
            pltpu.VMEM((_CHS, D), jnp.float32),
            pltpu.VMEM_SHARED((N, D), jnp.float32),
            pltpu.SemaphoreType.DMA,
            pltpu.SemaphoreType.DMA,
            pltpu.SemaphoreType.DMA,
            pltpu.SemaphoreType.DMA,
        ],
    )(_sc_scatter_body)
    return kfn(e_hat, receivers, zeros_nd)


_BE = 2000
_BN = 2000


def _dot(a, b):
    return jax.lax.dot_general(a, b, (((1,), (0,)), ((), ())),
                               preferred_element_type=jnp.float32)


def _full(shape):
    return pl.BlockSpec(shape, lambda i: (0,) * len(shape))


def _pass0_body(g_ref, rel_ref, w1_ref, b1_ref, w2_ref, b2_ref, out_ref):
    vs = g_ref[0]
    vr = g_ref[1]
    h = _dot(vs, w1_ref[0:D]) + _dot(vr, w1_ref[D:2 * D])
    h = h + _dot(rel_ref[...], w1_ref[2 * D:3 * D]) + b1_ref[...]
    h = jnp.maximum(h, 0.0)
    renc = _dot(h, w2_ref[...]) + b2_ref[...]
    out_ref[...] = _dot(renc, w1_ref[0:D]) + b1_ref[...]


def _pass0(g0, rel, re_W1, re_b1, re_W2, re_b2):
    return pl.pallas_call(
        _pass0_body,
        grid=(E // _BE,),
        in_specs=[
            pl.BlockSpec((2, _BE, D), lambda i: (0, i, 0)),
            pl.BlockSpec((_BE, D), lambda i: (i, 0)),
            _full((3 * D, H)),
            _full((1, H)),
            _full((H, D)),
            _full((1, D)),
        ],
        out_specs=pl.BlockSpec((_BE, H), lambda i: (i, 0)),
        out_shape=jax.ShapeDtypeStruct((E, H), jnp.float32),
    )(g0, rel, re_W1, re_b1.reshape(1, H), re_W2, re_b2.reshape(1, D))


def _edge_first_body(rp_ref, w2_ref, b2_ref, out_ref):
    h = jnp.maximum(rp_ref[...], 0.0)
    out_ref[...] = _dot(h, w2_ref[...]) + b2_ref[...]


def _edge_first(rel_part, re_W2, re_b2):
    return pl.pallas_call(
        _edge_first_body,
        grid=(E // _BE,),
        in_specs=[
            pl.BlockSpec((_BE, H), lambda i: (i, 0)),
            _full((H, D)),
            _full((1, D)),
        ],
        out_specs=pl.BlockSpec((_BE, D), lambda i: (i, 0)),
        out_shape=jax.ShapeDtypeStruct((E, D), jnp.float32),
    )(rel_part, re_W2, re_b2.reshape(1, D))


def _edge_body(rp_ref, g_ref, w1_ref, w2_ref, b2_ref, out_ref):
    h = rp_ref[...] + _dot(g_ref[0], w1_ref[0:D]) + _dot(g_ref[1], w1_ref[D:2 * D])
    h = jnp.maximum(h, 0.0)
    out_ref[...] = _dot(h, w2_ref[...]) + b2_ref[...]


def _edge(rel_part, g, re_W1, re_W2, re_b2):
    return pl.pallas_call(
        _edge_body,
        grid=(E // _BE,),
        in_specs=[
            pl.BlockSpec((_BE, H), lambda i: (i, 0)),
            pl.BlockSpec((2, _BE, D), lambda i: (0, i, 0)),
            _full((2 * D, H)),
            _full((H, D)),
            _full((1, D)),
        ],
        out_specs=pl.BlockSpec((_BE, D), lambda i: (i, 0)),
        out_shape=jax.ShapeDtypeStruct((E, D), jnp.float32),
    )(rel_part, g, re_W1[D:3 * D], re_W2, re_b2.reshape(1, D))


def _prologue_body(obj_ref, ow1_ref, ob1_ref, ow2_ref, ob2_ref,
                   ea_w1a_ref, ea_b1_ref, out_ref):
    h = jnp.maximum(_dot(obj_ref[...], ow1_ref[...]) + ob1_ref[...], 0.0)
    enc = _dot(h, ow2_ref[...]) + ob2_ref[...]
    out_ref[...] = _dot(enc, ea_w1a_ref[...]) + ea_b1_ref[...]


def _prologue(obj, oe_W1, oe_b1, oe_W2, oe_b2, ea_W1, ea_b1):
    return pl.pallas_call(
        _prologue_body,
        grid=(N // _BN,),
        in_specs=[
            pl.BlockSpec((_BN, D), lambda i: (i, 0)),
            _full((D, H)),
            _full((1, H)),
            _full((H, D)),
            _full((1, D)),
            _full((D, H)),
            _full((1, H)),
        ],
        out_specs=pl.BlockSpec((_BN, H), lambda i: (i, 0)),
        out_shape=jax.ShapeDtypeStruct((N, H), jnp.float32),
    )(obj, oe_W1, oe_b1.reshape(1, H), oe_W2, oe_b2.reshape(1, D),
      ea_W1[0:D], ea_b1.reshape(1, H))


def _node_body(op_ref, agg_ref, v_ref, w1_ref, w2_ref, b2_ref, out_ref,
               *, with_v):
    agg = agg_ref[0] + agg_ref[1]
    h = op_ref[...] + _dot(agg, w1_ref[0:D])
    if with_v:
        h = h + _dot(v_ref[...], w1_ref[D:2 * D])
    h = jnp.maximum(h, 0.0)
    out_ref[...] = _dot(h, w2_ref[...]) + b2_ref[...]


def _node(obj_part, agg, v, ea_W1, ea_W2, ea_b2, with_v):
    return pl.pallas_call(
        functools.partial(_node_body, with_v=with_v),
        grid=(N // _BN,),
        in_specs=[
            pl.BlockSpec((_BN, H), lambda i: (i, 0)),
            pl.BlockSpec((2, _BN, D), lambda i: (0, i, 0)),
            pl.BlockSpec((_BN, D), lambda i: (i, 0)),
            _full((2 * D, H)),
            _full((H, D)),
            _full((1, D)),
        ],
        out_specs=pl.BlockSpec((_BN, D), lambda i: (i, 0)),
        out_shape=jax.ShapeDtypeStruct((N, D), jnp.float32),
    )(obj_part, agg, v, ea_W1[D:3 * D], ea_W2, ea_b2.reshape(1, D))


def _decode_body(v_ref, w1_ref, b1_ref, w2_ref, b2_ref, out_ref):
    h = jnp.maximum(_dot(v_ref[...], w1_ref[...]) + b1_ref[...], 0.0)
    out_ref[...] = _dot(h, w2_ref[...]) + b2_ref[...]


def _decode(v, od_W1, od_b1, od_W2, od_b2):
    return pl.pallas_call(
        _decode_body,
        grid=(N // _BN,),
        in_specs=[
            pl.BlockSpec((_BN, D), lambda i: (i, 0)),
            _full((D, H)),
            _full((1, H)),
            _full((H, D)),
            _full((1, D)),
        ],
        out_specs=pl.BlockSpec((_BN, D), lambda i: (i, 0)),
        out_shape=jax.ShapeDtypeStruct((N, D), jnp.float32),
    )(v, od_W1, od_b1.reshape(1, H), od_W2, od_b2.reshape(1, D))


def kernel(objects, relations, senders, receivers,
           re_W1, re_b1, re_W2, re_b2,
           oe_W1, oe_b1, oe_W2, oe_b2,
           ea_W1, ea_b1, ea_W2, ea_b2,
           od_W1, od_b1, od_W2, od_b2):
    assert objects.shape == (1, N, D) and relations.shape == (1, E, D)
    obj = objects[0]
    rel = relations[0]

    obj_part = _prologue(obj, oe_W1, oe_b1, oe_W2, oe_b2, ea_W1, ea_b1)
    g0 = _sc_gather(obj, senders, receivers)
    rel_part = _pass0(g0, rel, re_W1, re_b1, re_W2, re_b2)

    zeros_nd = jnp.zeros((N, D), jnp.float32)

    e_hat = _edge_first(rel_part, re_W2, re_b2)
    agg = _sc_scatter(e_hat, receivers, zeros_nd)
    v = _node(obj_part, agg, zeros_nd, ea_W1, ea_W2, ea_b2, with_v=False)

    for _ in range(2):
        g = _sc_gather(v, senders, receivers)
        e_hat = _edge(rel_part, g, re_W1, re_W2, re_b2)
        agg = _sc_scatter(e_hat, receivers, zeros_nd)
        v = _node(obj_part, agg, v, ea_W1, ea_W2, ea_b2, with_v=True)

    out = _decode(v, od_W1, od_b1, od_W2, od_b2)
    return out[None]
